# Initial kernel scaffold; baseline (speedup 1.0000x reference)
import functools

import jax
import jax.numpy as jnp
from jax import lax
from jax.experimental import pallas as pl
from jax.experimental.pallas import tpu as pltpu
from jax.experimental.pallas import tpu_sc as plsc

NC = 2
NS = 16
L = 16


def _edge_alpha(edge_attr, wcat):
    E, ED = edge_attr.shape
    BR = 2000
    assert E % BR == 0

    def body(a_ref, w_ref, o_ref):
        o_ref[...] = jnp.dot(a_ref[...], w_ref[...],
                             preferred_element_type=jnp.float32)

    return pl.pallas_call(
        body,
        grid=(E // BR,),
        in_specs=[pl.BlockSpec((BR, ED), lambda i: (i, 0)),
                  pl.BlockSpec((ED, 8), lambda i: (0, 0))],
        out_specs=pl.BlockSpec((BR, 8), lambda i: (i, 0)),
        out_shape=jax.ShapeDtypeStruct((E, 8), jnp.float32),
    )(edge_attr, wcat)


def _node_proj(h_in, W, A, relu_bias=None):
    NPAD, Din = h_in[0].shape if isinstance(h_in, tuple) else h_in.shape
    H = W.shape[1]
    BN = 640
    assert NPAD % BN == 0

    if isinstance(h_in, tuple):
        p0, p1 = h_in
        b = relu_bias

        def body2(p0_ref, p1_ref, b_ref, w_ref, a_ref, h_ref, asd_ref):
            hv = jax.nn.relu(p0_ref[...] + p1_ref[...] + b_ref[...])
            h = jnp.dot(hv, w_ref[...], preferred_element_type=jnp.float32)
            h_ref[...] = h
            asd_ref[...] = jnp.dot(h, a_ref[...],
                                   preferred_element_type=jnp.float32)

        return pl.pallas_call(
            body2,
            grid=(NPAD // BN,),
            in_specs=[pl.BlockSpec((BN, Din), lambda i: (i, 0)),
                      pl.BlockSpec((BN, Din), lambda i: (i, 0)),
                      pl.BlockSpec((1, Din), lambda i: (0, 0)),
                      pl.BlockSpec((Din, H), lambda i: (0, 0)),
                      pl.BlockSpec((H, 8), lambda i: (0, 0))],
            out_specs=[pl.BlockSpec((BN, H), lambda i: (i, 0)),
                       pl.BlockSpec((BN, 8), lambda i: (i, 0))],
            out_shape=[jax.ShapeDtypeStruct((NPAD, H), jnp.float32),
                       jax.ShapeDtypeStruct((NPAD, 8), jnp.float32)],
        )(p0, p1, b, W, A)

    def body1(h_ref, w_ref, a_ref, hp_ref, asd_ref):
        h = jnp.dot(h_ref[...], w_ref[...], preferred_element_type=jnp.float32)
        hp_ref[...] = h
        asd_ref[...] = jnp.dot(h, a_ref[...], preferred_element_type=jnp.float32)

    return pl.pallas_call(
        body1,
        grid=(NPAD // BN,),
        in_specs=[pl.BlockSpec((BN, Din), lambda i: (i, 0)),
                  pl.BlockSpec((Din, H), lambda i: (0, 0)),
                  pl.BlockSpec((H, 8), lambda i: (0, 0))],
        out_specs=[pl.BlockSpec((BN, H), lambda i: (i, 0)),
                   pl.BlockSpec((BN, 8), lambda i: (i, 0))],
        out_shape=[jax.ShapeDtypeStruct((NPAD, H), jnp.float32),
                   jax.ShapeDtypeStruct((NPAD, 8), jnp.float32)],
    )(h_in, W, A)


def _combine(p0, p1, b):
    NPAD, F = p0.shape
    BN = 640

    def body(p0_ref, p1_ref, b_ref, o_ref):
        o_ref[...] = p0_ref[...] + p1_ref[...] + b_ref[...]

    return pl.pallas_call(
        body,
        grid=(NPAD // BN,),
        in_specs=[pl.BlockSpec((BN, F), lambda i: (i, 0)),
                  pl.BlockSpec((BN, F), lambda i: (i, 0)),
                  pl.BlockSpec((1, F), lambda i: (0, 0))],
        out_specs=pl.BlockSpec((BN, F), lambda i: (i, 0)),
        out_shape=jax.ShapeDtypeStruct((NPAD, F), jnp.float32),
    )(p0, p1, b)


def _sc_gat_layer(src2d, dst2d, ae2d, asp, adp, h_tab, z1, z2, F, NPAD, TR):
    HR = TR // NC
    NZ = NPAD // NS
    mesh = plsc.VectorSubcoreMesh(core_axis_name="c", subcore_axis_name="s")

    @functools.partial(
        pl.kernel,
        out_type=jax.ShapeDtypeStruct((NC * NPAD, F), jnp.float32),
        mesh=mesh,
        scratch_types=[
            pltpu.VMEM((TR, 128), jnp.int32),
            pltpu.VMEM((TR, 128), jnp.int32),
            pltpu.VMEM((TR, 128), jnp.float32),
            pltpu.VMEM((TR, 128), jnp.float32),
            pltpu.VMEM((NPAD,), jnp.float32),
            pltpu.VMEM((NPAD,), jnp.float32),
            pltpu.VMEM((NPAD,), jnp.float32),
            pltpu.VMEM((128,), jnp.float32),
            pltpu.VMEM((128, F), jnp.float32),
            pltpu.VMEM_SHARED((NPAD,), jnp.float32),
            pltpu.VMEM_SHARED((NPAD, F), jnp.float32),
            pltpu.SemaphoreType.DMA,
        ],
    )
    def k(src_h, dst_h, ae_h, as_h, ad_h, ht_h, z1_h, z2_h, out_h,
          src_v, dst_v, ae_v, ex_v, as_v, ad_v, den_v, coef_v, hbuf,
          den_sh, acc_sh, sem):
        c = lax.axis_index("c")
        s = lax.axis_index("s")

        pltpu.sync_copy(z1_h.at[pl.ds(s * NZ, NZ)], den_sh.at[pl.ds(s * NZ, NZ)])
        pltpu.sync_copy(z2_h.at[pl.ds(s * NZ, NZ)], acc_sh.at[pl.ds(s * NZ, NZ)])

        r0 = s * TR
        pltpu.sync_copy(src_h.at[pl.ds(r0, TR)], src_v)
        pltpu.sync_copy(dst_h.at[pl.ds(r0, TR)], dst_v)
        pltpu.sync_copy(ae_h.at[pl.ds(r0, TR)], ae_v)
        pltpu.sync_copy(as_h, as_v)
        pltpu.sync_copy(ad_h, ad_v)

        def exp_row(j, carry):
            for o in range(128 // L):
                sl = pl.ds(o * L, L)
                lg = (ae_v[j, sl]
                      + plsc.load_gather(as_v, [src_v[j, sl]])
                      + plsc.load_gather(ad_v, [dst_v[j, sl]]))
                lg = jnp.maximum(lg, 0.2 * lg)
                ex_v[j, sl] = jnp.exp(lg)
            return carry
        lax.fori_loop(0, TR, exp_row, 0)
        plsc.subcore_barrier()

        def den_row(j, carry):
            pltpu.sync_copy(ex_v.at[j], den_sh.at[dst_v.at[j]], add=True)
            return carry
        lax.fori_loop(0, TR, den_row, 0)
        plsc.subcore_barrier()
        pltpu.sync_copy(den_sh, den_v)

        def agg_row(j2, carry):
            j = c * HR + j2
            pltpu.async_copy(ht_h.at[src_v.at[j]], hbuf, sem).wait()
            for o in range(128 // L):
                sl = pl.ds(o * L, L)
                dden = plsc.load_gather(den_v, [dst_v[j, sl]])
                coef_v[sl] = ex_v[j, sl] / (dden + 1e-16)

            def scale_edge(e, icarry):
                cf = coef_v[e]
                for q in range(F // L):
                    qs = pl.ds(q * L, L)
                    hbuf[e, qs] = hbuf[e, qs] * cf
                return icarry
            lax.fori_loop(0, 128, scale_edge, 0)
            pltpu.sync_copy(hbuf, acc_sh.at[dst_v.at[j]], add=True)
            return carry
        lax.fori_loop(0, HR, agg_row, 0)
        plsc.subcore_barrier()

        pltpu.sync_copy(acc_sh.at[pl.ds(s * NZ, NZ)],
                        out_h.at[pl.ds(c * NPAD + s * NZ, NZ)])

    return k(src2d, dst2d, ae2d, asp, adp, h_tab, z1, z2)


def kernel(x, edge_index, edge_attr, emb, W1, att_src1, att_dst1, We1,
           att_edge1, b1, W2, att_src2, att_dst2, We2, att_edge2, b2):
    N, D = emb.shape
    E = edge_index.shape[1]
    ED = edge_attr.shape[1]
    H1 = W1.shape[1]
    OUT = W2.shape[1]

    NPAD = -(-(N + 1) // 640) * 640
    TR = -(-E // (NS * 128 * NC)) * NC
    EP = TR * NS * 128

    pad_e = EP - E
    src_p = jnp.concatenate([edge_index[0], jnp.zeros((pad_e,), jnp.int32)])
    dst_p = jnp.concatenate([edge_index[1], jnp.full((pad_e,), N, jnp.int32)])
    src2d = src_p.reshape(TR * NS, 128)
    dst2d = dst_p.reshape(TR * NS, 128)

    we1 = We1 @ att_edge1
    we2 = We2 @ att_edge2
    wcat = jnp.zeros((ED, 8), jnp.float32).at[:, 0].set(we1).at[:, 1].set(we2)
    ae8 = _edge_alpha(edge_attr, wcat)
    ae_pad = jnp.full((pad_e,), -1e30, jnp.float32)
    ae1_2d = jnp.concatenate([ae8[:, 0], ae_pad]).reshape(TR * NS, 128)
    ae2_2d = jnp.concatenate([ae8[:, 1], ae_pad]).reshape(TR * NS, 128)

    h0 = jnp.take(emb, x, axis=0)
    h0p = jnp.zeros((NPAD, D), jnp.float32).at[:N].set(h0)

    A1 = jnp.zeros((H1, 8), jnp.float32).at[:, 0].set(att_src1).at[:, 1].set(att_dst1)
    A2 = jnp.zeros((OUT, 8), jnp.float32).at[:, 0].set(att_src2).at[:, 1].set(att_dst2)

    z1 = jnp.zeros((NPAD,), jnp.float32)
    zH1 = jnp.zeros((NPAD, H1), jnp.float32)
    zH2 = jnp.zeros((NPAD, OUT), jnp.float32)

    h1pre, asd1 = _node_proj(h0p, W1, A1)
    p1 = _sc_gat_layer(src2d, dst2d, ae1_2d,
                       jnp.ascontiguousarray(asd1[:, 0]),
                       jnp.ascontiguousarray(asd1[:, 1]),
                       h1pre, z1, zH1, H1, NPAD, TR)

    h2pre, asd2 = _node_proj((p1[:NPAD], p1[NPAD:]), W2, A2,
                             relu_bias=b1.reshape(1, H1))
    p2 = _sc_gat_layer(src2d, dst2d, ae2_2d,
                       jnp.ascontiguousarray(asd2[:, 0]),
                       jnp.ascontiguousarray(asd2[:, 1]),
                       h2pre, z1, zH2, OUT, NPAD, TR)

    out = _combine(p2[:NPAD], p2[NPAD:], b2.reshape(1, OUT))
    return out[:N]


# speedup vs baseline: 13.0722x; 13.0722x over previous
import functools

import jax
import jax.numpy as jnp
from jax import lax
from jax.experimental import pallas as pl
from jax.experimental.pallas import tpu as pltpu
from jax.experimental.pallas import tpu_sc as plsc

NC = 2
NS = 16
L = 16


def _edge_alpha(edge_attr, wcat):
    E, ED = edge_attr.shape
    BR = 2000 if E % 2000 == 0 else E

    def body(a_ref, w_ref, o_ref):
        o_ref[...] = jnp.dot(a_ref[...], w_ref[...],
                             preferred_element_type=jnp.float32)

    return pl.pallas_call(
        body,
        grid=(E // BR,),
        in_specs=[pl.BlockSpec((BR, ED), lambda i: (i, 0)),
                  pl.BlockSpec((ED, 8), lambda i: (0, 0))],
        out_specs=pl.BlockSpec((BR, 8), lambda i: (i, 0)),
        out_shape=jax.ShapeDtypeStruct((E, 8), jnp.float32),
    )(edge_attr, wcat)


def _node_proj(h_in, W, A, relu_bias=None):
    NPAD, Din = h_in[0].shape if isinstance(h_in, tuple) else h_in.shape
    H = W.shape[1]
    BN = 640
    assert NPAD % BN == 0

    if isinstance(h_in, tuple):
        p0, p1, d0, d1 = h_in
        b = relu_bias

        def body2(p0_ref, p1_ref, d0_ref, d1_ref, b_ref, w_ref, a_ref,
                  h_ref, asd_ref):
            den = d0_ref[...] + d1_ref[...] + 1e-16
            hv = jax.nn.relu((p0_ref[...] + p1_ref[...]) / den + b_ref[...])
            h = jnp.dot(hv, w_ref[...], preferred_element_type=jnp.float32)
            h_ref[...] = h
            asd_ref[...] = jnp.dot(h, a_ref[...],
                                   preferred_element_type=jnp.float32)

        return pl.pallas_call(
            body2,
            grid=(NPAD // BN,),
            in_specs=[pl.BlockSpec((BN, Din), lambda i: (i, 0)),
                      pl.BlockSpec((BN, Din), lambda i: (i, 0)),
                      pl.BlockSpec((BN, 1), lambda i: (i, 0)),
                      pl.BlockSpec((BN, 1), lambda i: (i, 0)),
                      pl.BlockSpec((1, Din), lambda i: (0, 0)),
                      pl.BlockSpec((Din, H), lambda i: (0, 0)),
                      pl.BlockSpec((H, 8), lambda i: (0, 0))],
            out_specs=[pl.BlockSpec((BN, H), lambda i: (i, 0)),
                       pl.BlockSpec((BN, 8), lambda i: (i, 0))],
            out_shape=[jax.ShapeDtypeStruct((NPAD, H), jnp.float32),
                       jax.ShapeDtypeStruct((NPAD, 8), jnp.float32)],
        )(p0, p1, d0, d1, b, W, A)

    def body1(h_ref, w_ref, a_ref, hp_ref, asd_ref):
        h = jnp.dot(h_ref[...], w_ref[...], preferred_element_type=jnp.float32)
        hp_ref[...] = h
        asd_ref[...] = jnp.dot(h, a_ref[...], preferred_element_type=jnp.float32)

    return pl.pallas_call(
        body1,
        grid=(NPAD // BN,),
        in_specs=[pl.BlockSpec((BN, Din), lambda i: (i, 0)),
                  pl.BlockSpec((Din, H), lambda i: (0, 0)),
                  pl.BlockSpec((H, 8), lambda i: (0, 0))],
        out_specs=[pl.BlockSpec((BN, H), lambda i: (i, 0)),
                   pl.BlockSpec((BN, 8), lambda i: (i, 0))],
        out_shape=[jax.ShapeDtypeStruct((NPAD, H), jnp.float32),
                   jax.ShapeDtypeStruct((NPAD, 8), jnp.float32)],
    )(h_in, W, A)


def _combine(p0, p1, d0, d1, b):
    NPAD, F = p0.shape
    BN = 640

    def body(p0_ref, p1_ref, d0_ref, d1_ref, b_ref, o_ref):
        den = d0_ref[...] + d1_ref[...] + 1e-16
        o_ref[...] = (p0_ref[...] + p1_ref[...]) / den + b_ref[...]

    return pl.pallas_call(
        body,
        grid=(NPAD // BN,),
        in_specs=[pl.BlockSpec((BN, F), lambda i: (i, 0)),
                  pl.BlockSpec((BN, F), lambda i: (i, 0)),
                  pl.BlockSpec((BN, 1), lambda i: (i, 0)),
                  pl.BlockSpec((BN, 1), lambda i: (i, 0)),
                  pl.BlockSpec((1, F), lambda i: (0, 0))],
        out_specs=pl.BlockSpec((BN, F), lambda i: (i, 0)),
        out_shape=jax.ShapeDtypeStruct((NPAD, F), jnp.float32),
    )(p0, p1, d0, d1, b)


def _gather_rows(tab_h, idx_ref, out_ref, sem):
    pltpu.async_copy(tab_h.at[idx_ref], out_ref, sem).wait()


def _scatter_add(val_ref, tab_ref, idx_ref):
    pltpu.sync_copy(val_ref, tab_ref.at[idx_ref], add=True)


def _sc_gat_layer(src2d, dst2d, ae2d, asp, adp, h_tab, z1, z2, F, NPAD, TW):
    NZ = NPAD // NS
    mesh = plsc.VectorSubcoreMesh(core_axis_name="c", subcore_axis_name="s",
                                  num_cores=NC, num_subcores=NS)

    @functools.partial(
        pl.kernel,
        out_type=[jax.ShapeDtypeStruct((NC * NPAD, F), jnp.float32),
                  jax.ShapeDtypeStruct((NC * NPAD,), jnp.float32)],
        mesh=mesh,
        compiler_params=pltpu.CompilerParams(needs_layout_passes=False,
                                             use_tc_tiling_on_sc=False),
        scratch_types=[
            pltpu.VMEM((TW, 64), jnp.int32),
            pltpu.VMEM((TW, 64), jnp.int32),
            pltpu.VMEM((TW, 64), jnp.float32),
            pltpu.VMEM((NPAD,), jnp.float32),
            pltpu.VMEM((NPAD,), jnp.float32),
            pltpu.VMEM((64, F), jnp.float32),
            pltpu.VMEM_SHARED((NPAD,), jnp.float32),
            pltpu.VMEM_SHARED((NPAD, F), jnp.float32),
            pltpu.SemaphoreType.DMA,
        ],
    )
    def k(src_h, dst_h, ae_h, as_h, ad_h, ht_h, z1_h, z2_h, acc_out, den_out,
          src_v, dst_v, ex_v, as_v, ad_v, hbuf, den_sh, acc_sh, sem):
        c = lax.axis_index("c")
        s = lax.axis_index("s")

        pltpu.sync_copy(z1_h.at[pl.ds(s * NZ, NZ)], den_sh.at[pl.ds(s * NZ, NZ)])
        pltpu.sync_copy(z2_h.at[pl.ds(s * NZ, NZ)], acc_sh.at[pl.ds(s * NZ, NZ)])

        r0 = (c * NS + s) * TW
        pltpu.sync_copy(src_h.at[pl.ds(r0, TW)], src_v)
        pltpu.sync_copy(dst_h.at[pl.ds(r0, TW)], dst_v)
        pltpu.sync_copy(ae_h.at[pl.ds(r0, TW)], ex_v)
        pltpu.sync_copy(as_h, as_v)
        pltpu.sync_copy(ad_h, ad_v)

        def exp_row(j, carry):
            for o in range(64 // L):
                sl = pl.ds(o * L, L)
                lg = (ex_v[j, sl]
                      + plsc.load_gather(as_v, [src_v[j, sl]])
                      + plsc.load_gather(ad_v, [dst_v[j, sl]]))
                lg = jnp.maximum(lg, 0.2 * lg)
                ex_v[j, sl] = jnp.exp(lg)
            return carry
        lax.fori_loop(0, TW, exp_row, 0)
        plsc.subcore_barrier()

        def agg_row(j, carry):
            _scatter_add(ex_v.at[j], den_sh, dst_v.at[j])
            _gather_rows(ht_h, src_v.at[j], hbuf, sem)

            def scale_grp(o, icarry):
                exs = ex_v[j, pl.ds(o * L, L)]
                for e_ in range(L):
                    cf = exs[e_]
                    e = o * L + e_
                    for q in range(F // L):
                        qs = pl.ds(q * L, L)
                        hbuf[e, qs] = hbuf[e, qs] * cf
                return icarry
            lax.fori_loop(0, 64 // L, scale_grp, 0)
            _scatter_add(hbuf, acc_sh, dst_v.at[j])
            return carry
        lax.fori_loop(0, TW, agg_row, 0)
        plsc.subcore_barrier()

        pltpu.sync_copy(acc_sh.at[pl.ds(s * NZ, NZ)],
                        acc_out.at[pl.ds(c * NPAD + s * NZ, NZ)])
        pltpu.sync_copy(den_sh.at[pl.ds(s * NZ, NZ)],
                        den_out.at[pl.ds(c * NPAD + s * NZ, NZ)])

    return k(src2d, dst2d, ae2d, asp, adp, h_tab, z1, z2)


def kernel(x, edge_index, edge_attr, emb, W1, att_src1, att_dst1, We1,
           att_edge1, b1, W2, att_src2, att_dst2, We2, att_edge2, b2):
    N, D = emb.shape
    E = edge_index.shape[1]
    ED = edge_attr.shape[1]
    H1 = W1.shape[1]
    OUT = W2.shape[1]

    NPAD = -(-(N + 1) // 640) * 640
    TW = -(-E // (NC * NS * 64))
    EP = TW * NC * NS * 64

    pad_e = EP - E
    src_p = jnp.concatenate([edge_index[0], jnp.zeros((pad_e,), jnp.int32)])
    dst_p = jnp.concatenate([edge_index[1], jnp.full((pad_e,), N, jnp.int32)])
    src2d = src_p.reshape(EP // 64, 64)
    dst2d = dst_p.reshape(EP // 64, 64)

    we1 = We1 @ att_edge1
    we2 = We2 @ att_edge2
    wcat = jnp.zeros((ED, 8), jnp.float32).at[:, 0].set(we1).at[:, 1].set(we2)
    ae8 = _edge_alpha(edge_attr, wcat)
    ae_pad = jnp.full((pad_e,), -1e30, jnp.float32)
    ae1_2d = jnp.concatenate([ae8[:, 0], ae_pad]).reshape(EP // 64, 64)
    ae2_2d = jnp.concatenate([ae8[:, 1], ae_pad]).reshape(EP // 64, 64)

    h0 = jnp.take(emb, x, axis=0)
    h0p = jnp.zeros((NPAD, D), jnp.float32).at[:N].set(h0)

    A1 = jnp.zeros((H1, 8), jnp.float32).at[:, 0].set(att_src1).at[:, 1].set(att_dst1)
    A2 = jnp.zeros((OUT, 8), jnp.float32).at[:, 0].set(att_src2).at[:, 1].set(att_dst2)

    z1 = jnp.zeros((NPAD,), jnp.float32)
    zH1 = jnp.zeros((NPAD, H1), jnp.float32)
    zH2 = jnp.zeros((NPAD, OUT), jnp.float32)

    h1pre, asd1 = _node_proj(h0p, W1, A1)
    p1, den1 = _sc_gat_layer(src2d, dst2d, ae1_2d,
                             asd1[:, 0],
                             asd1[:, 1],
                             h1pre, z1, zH1, H1, NPAD, TW)
    d1a = den1[:NPAD].reshape(NPAD, 1)
    d1b = den1[NPAD:].reshape(NPAD, 1)

    h2pre, asd2 = _node_proj((p1[:NPAD], p1[NPAD:], d1a, d1b), W2, A2,
                             relu_bias=b1.reshape(1, H1))
    p2, den2 = _sc_gat_layer(src2d, dst2d, ae2_2d,
                             asd2[:, 0],
                             asd2[:, 1],
                             h2pre, z1, zH2, OUT, NPAD, TW)

    out = _combine(p2[:NPAD], p2[NPAD:],
                   den2[:NPAD].reshape(NPAD, 1), den2[NPAD:].reshape(NPAD, 1),
                   b2.reshape(1, OUT))
    return out[:N]
